# Initial kernel scaffold; baseline (speedup 1.0000x reference)
import jax
import jax.numpy as jnp
from jax.experimental import pallas as pl


def kernel(x, A, W1, b1, W2, b2):
    raise NotImplementedError("write your pallas kernel here")


# speedup vs baseline: 30.4950x; 30.4950x over previous
import functools

import jax
import jax.numpy as jnp
from jax import lax
from jax.experimental import pallas as pl
from jax.experimental.pallas import tpu as pltpu
from jax.experimental.pallas import tpu_sc as plsc

N = 10000
E = 320000
IN_C = 128
HID = 16
OUT_C = 128

NC = 2
NS = 16
NW = NC * NS
CH = 128
NCH = -(-E // (NW * CH))
EPT = NCH * CH
EPAD = EPT * NW
NPAD = 10240
RPT = NPAD // NS

_MESH = plsc.VectorSubcoreMesh(
    core_axis_name="c", subcore_axis_name="s", num_cores=NC, num_subcores=NS)

_SC_PARAMS = pltpu.CompilerParams(use_tc_tiling_on_sc=False)


def _zero_stage(stage_v, nrows):
    z = jnp.zeros((16,), jnp.float32)

    @pl.loop(0, nrows)
    def _(i):
        stage_v[i, :] = z


def _deg_body(dst_hbm, ones_hbm, out_hbm, dstv, ones_v, stage_v, acc, sem):
    cid = lax.axis_index("c")
    sid = lax.axis_index("s")
    tid = cid * NS + sid

    pltpu.sync_copy(dst_hbm.at[tid], dstv)
    pltpu.sync_copy(ones_hbm, ones_v)
    _zero_stage(stage_v, RPT)
    pltpu.sync_copy(stage_v, acc.at[pl.ds(sid * RPT, RPT)])
    plsc.subcore_barrier()

    @pl.loop(0, NCH)
    def _(c):
        pltpu.sync_copy(ones_v, acc.at[dstv.at[c]], add=True)

    plsc.subcore_barrier()
    pltpu.sync_copy(acc.at[pl.ds(sid * RPT, RPT)], stage_v)
    pltpu.sync_copy(stage_v, out_hbm.at[cid, pl.ds(sid * RPT, RPT)])


_deg_call = pl.kernel(
    _deg_body,
    out_type=jax.ShapeDtypeStruct((NC, NPAD, HID), jnp.float32),
    mesh=_MESH,
    scratch_types=[
        pltpu.VMEM((NCH, CH), jnp.int32),
        pltpu.VMEM((CH, HID), jnp.float32),
        pltpu.VMEM((RPT, HID), jnp.float32),
        pltpu.VMEM_SHARED((NPAD, HID), jnp.float32),
        pltpu.SemaphoreType.DMA,
    ],
    compiler_params=_SC_PARAMS,
)


def _agg_body(g_hbm, src_hbm, dst_hbm, out_hbm,
              srcv, dstv, rows_v, stage_v, acc, sem):
    cid = lax.axis_index("c")
    sid = lax.axis_index("s")
    tid = cid * NS + sid

    pltpu.sync_copy(src_hbm.at[tid], srcv)
    pltpu.sync_copy(dst_hbm.at[tid], dstv)
    _zero_stage(stage_v, RPT)
    pltpu.sync_copy(stage_v, acc.at[pl.ds(sid * RPT, RPT)])
    plsc.subcore_barrier()

    @pl.loop(0, NCH)
    def _(c):
        pltpu.async_copy(g_hbm.at[srcv.at[c]], rows_v, sem).wait()
        pltpu.sync_copy(rows_v, acc.at[dstv.at[c]], add=True)

    plsc.subcore_barrier()
    pltpu.sync_copy(acc.at[pl.ds(sid * RPT, RPT)], stage_v)
    pltpu.sync_copy(stage_v, out_hbm.at[cid, pl.ds(sid * RPT, RPT)])


_agg_call = pl.kernel(
    _agg_body,
    out_type=jax.ShapeDtypeStruct((NC, NPAD, HID), jnp.float32),
    mesh=_MESH,
    scratch_types=[
        pltpu.VMEM((NCH, CH), jnp.int32),
        pltpu.VMEM((NCH, CH), jnp.int32),
        pltpu.VMEM((CH, HID), jnp.float32),
        pltpu.VMEM((RPT, HID), jnp.float32),
        pltpu.VMEM_SHARED((NPAD, HID), jnp.float32),
        pltpu.SemaphoreType.DMA,
    ],
    compiler_params=_SC_PARAMS,
)


def _tc1_body(x_ref, w1_ref, d0_ref, d1_ref, g1_ref, dinv_ref):
    deg = d0_ref[...] + d1_ref[...] + 1.0
    dinv = lax.rsqrt(deg)
    h = jnp.dot(x_ref[...], w1_ref[...], preferred_element_type=jnp.float32)
    g1_ref[...] = h * dinv
    dinv_ref[...] = dinv


def _tc2_body(g1_ref, p0_ref, p1_ref, dinv_ref, b1_ref, g2_ref):
    s = g1_ref[...] + p0_ref[...] + p1_ref[...]
    h1 = jnp.maximum(s * dinv_ref[...] + b1_ref[...], 0.0)
    g2_ref[...] = h1 * dinv_ref[...]


def _tc3_body(g2_ref, q0_ref, q1_ref, dinv_ref, w2_ref, b2_ref, out_ref):
    s = (g2_ref[...] + q0_ref[...] + q1_ref[...]) * dinv_ref[...]
    out_ref[...] = (
        jnp.dot(s, w2_ref[...], preferred_element_type=jnp.float32)
        + b2_ref[...])


_tc1_call = pl.pallas_call(
    _tc1_body,
    out_shape=(jax.ShapeDtypeStruct((N, HID), jnp.float32),
               jax.ShapeDtypeStruct((N, 1), jnp.float32)),
)

_tc2_call = pl.pallas_call(
    _tc2_body,
    out_shape=jax.ShapeDtypeStruct((N, HID), jnp.float32),
)

_tc3_call = pl.pallas_call(
    _tc3_body,
    out_shape=jax.ShapeDtypeStruct((N, OUT_C), jnp.float32),
)


def kernel(x, A, W1, b1, W2, b2):
    src = A[0].astype(jnp.int32)
    dst = A[1].astype(jnp.int32)
    pad = EPAD - E
    srcp = jnp.concatenate(
        [src, jnp.zeros((pad,), jnp.int32)]).reshape(NW, NCH, CH)
    dstp = jnp.concatenate(
        [dst, jnp.full((pad,), N, jnp.int32)]).reshape(NW, NCH, CH)
    ones_rows = jnp.ones((CH, HID), jnp.float32)

    degp = _deg_call(dstp, ones_rows)
    d0 = degp[0, :N, 0:1]
    d1 = degp[1, :N, 0:1]

    g1, dinv = _tc1_call(x, W1, d0, d1)
    p = _agg_call(g1, srcp, dstp)
    g2 = _tc2_call(g1, p[0, :N], p[1, :N], dinv, b1.reshape(1, HID))
    q = _agg_call(g2, srcp, dstp)
    out = _tc3_call(g2, q[0, :N], q[1, :N], dinv, W2, b2.reshape(1, OUT_C))
    return out


# speedup vs baseline: 36.1925x; 1.1868x over previous
import jax
import jax.numpy as jnp
from jax import lax
from jax.experimental import pallas as pl
from jax.experimental.pallas import tpu as pltpu
from jax.experimental.pallas import tpu_sc as plsc

N = 10000
E = 320000
IN_C = 128
HID = 16
OUT_C = 128

NC = 2
NS = 16
NW = NC * NS
CH = 128
NBUF = 8
NCH = 80
EPT = NCH * CH
EPAD = EPT * NW
NPAD = 10240
RPT = NPAD // NS

_MESH = plsc.VectorSubcoreMesh(
    core_axis_name="c", subcore_axis_name="s", num_cores=NC, num_subcores=NS)

_SC_PARAMS = pltpu.CompilerParams(use_tc_tiling_on_sc=False)


def _zero_stage(stage_v, nrows):
    z = jnp.zeros((16,), jnp.float32)

    @pl.loop(0, nrows)
    def _(i):
        stage_v[i, :] = z


def _deg_body(dst_hbm, ones_hbm, out_hbm, dstv, ones_v, stage_v, acc, sem):
    cid = lax.axis_index("c")
    sid = lax.axis_index("s")
    tid = cid * NS + sid

    pltpu.sync_copy(dst_hbm.at[tid], dstv)
    pltpu.sync_copy(ones_hbm, ones_v)
    _zero_stage(stage_v, RPT)
    pltpu.sync_copy(stage_v, acc.at[pl.ds(sid * RPT, RPT)])
    plsc.subcore_barrier()

    @pl.loop(0, NCH // NBUF)
    def _(i):
        for b in range(NBUF):
            pltpu.async_copy(ones_v, acc.at[dstv.at[i * NBUF + b]], sem,
                             add=True)
        for b in range(NBUF):
            pltpu.make_async_copy(
                ones_v, acc.at[dstv.at[i * NBUF + b]], sem).wait()

    plsc.subcore_barrier()
    pltpu.sync_copy(acc.at[pl.ds(sid * RPT, RPT)],
                    out_hbm.at[cid, pl.ds(sid * RPT, RPT)])


_deg_call = pl.kernel(
    _deg_body,
    out_type=jax.ShapeDtypeStruct((NC, NPAD, HID), jnp.float32),
    mesh=_MESH,
    scratch_types=[
        pltpu.VMEM((NCH, CH), jnp.int32),
        pltpu.VMEM((CH, HID), jnp.float32),
        pltpu.VMEM((RPT, HID), jnp.float32),
        pltpu.VMEM_SHARED((NPAD, HID), jnp.float32),
        pltpu.SemaphoreType.DMA,
    ],
    compiler_params=_SC_PARAMS,
)


def _agg_body(g_hbm, src_hbm, dst_hbm, out_hbm,
              srcv, dstv, rows, stage_v, acc, *sems):
    gsems = sems[:NBUF]
    ssems = sems[NBUF:]
    cid = lax.axis_index("c")
    sid = lax.axis_index("s")
    tid = cid * NS + sid

    pltpu.sync_copy(src_hbm.at[tid], srcv)
    pltpu.sync_copy(dst_hbm.at[tid], dstv)
    _zero_stage(stage_v, RPT)
    pltpu.sync_copy(stage_v, acc.at[pl.ds(sid * RPT, RPT)])
    plsc.subcore_barrier()

    for b in range(NBUF):
        pltpu.async_copy(g_hbm.at[srcv.at[b]], rows.at[b], gsems[b])

    niter = NCH // NBUF

    @pl.loop(0, niter)
    def _(i):
        not_last = i + 1 < niter
        for b in range(NBUF):
            c = i * NBUF + b
            pltpu.make_async_copy(
                g_hbm.at[srcv.at[c]], rows.at[b], gsems[b]).wait()
            pltpu.async_copy(rows.at[b], acc.at[dstv.at[c]], ssems[b],
                             add=True)
        for b in range(NBUF):
            c = i * NBUF + b
            pltpu.make_async_copy(
                rows.at[b], acc.at[dstv.at[c]], ssems[b]).wait()

            @pl.when(not_last)
            def _():
                pltpu.async_copy(
                    g_hbm.at[srcv.at[c + NBUF]], rows.at[b], gsems[b])

    plsc.subcore_barrier()
    pltpu.sync_copy(acc.at[pl.ds(sid * RPT, RPT)],
                    out_hbm.at[cid, pl.ds(sid * RPT, RPT)])


_agg_call = pl.kernel(
    _agg_body,
    out_type=jax.ShapeDtypeStruct((NC, NPAD, HID), jnp.float32),
    mesh=_MESH,
    scratch_types=[
        pltpu.VMEM((NCH, CH), jnp.int32),
        pltpu.VMEM((NCH, CH), jnp.int32),
        pltpu.VMEM((NBUF, CH, HID), jnp.float32),
        pltpu.VMEM((RPT, HID), jnp.float32),
        pltpu.VMEM_SHARED((NPAD, HID), jnp.float32),
    ] + [pltpu.SemaphoreType.DMA] * (2 * NBUF),
    compiler_params=_SC_PARAMS,
)


def _tc0_body(x_ref, w1_ref, h_ref):
    h_ref[...] = jnp.dot(x_ref[...], w1_ref[...],
                         preferred_element_type=jnp.float32)


def _tc1_body(h_ref, d0_ref, d1_ref, g1_ref, dinv_ref):
    deg = d0_ref[...] + d1_ref[...] + 1.0
    dinv = lax.rsqrt(deg)
    g1_ref[...] = h_ref[...] * dinv
    dinv_ref[...] = dinv


def _tc2_body(g1_ref, p0_ref, p1_ref, dinv_ref, b1_ref, g2_ref):
    s = g1_ref[...] + p0_ref[...] + p1_ref[...]
    h1 = jnp.maximum(s * dinv_ref[...] + b1_ref[...], 0.0)
    g2_ref[...] = h1 * dinv_ref[...]


def _tc3_body(g2_ref, q0_ref, q1_ref, dinv_ref, w2_ref, b2_ref, out_ref):
    s = (g2_ref[...] + q0_ref[...] + q1_ref[...]) * dinv_ref[...]
    out_ref[...] = (
        jnp.dot(s, w2_ref[...], preferred_element_type=jnp.float32)
        + b2_ref[...])


_tc0_call = pl.pallas_call(
    _tc0_body,
    out_shape=jax.ShapeDtypeStruct((N, HID), jnp.float32),
)

_tc1_call = pl.pallas_call(
    _tc1_body,
    out_shape=(jax.ShapeDtypeStruct((N, HID), jnp.float32),
               jax.ShapeDtypeStruct((N, 1), jnp.float32)),
)

_tc2_call = pl.pallas_call(
    _tc2_body,
    out_shape=jax.ShapeDtypeStruct((N, HID), jnp.float32),
)

_tc3_call = pl.pallas_call(
    _tc3_body,
    out_shape=jax.ShapeDtypeStruct((N, OUT_C), jnp.float32),
)


def kernel(x, A, W1, b1, W2, b2):
    src = A[0].astype(jnp.int32)
    dst = A[1].astype(jnp.int32)
    pad = EPAD - E
    srcp = jnp.concatenate(
        [src, jnp.zeros((pad,), jnp.int32)]).reshape(NW, NCH, CH)
    dstp = jnp.concatenate(
        [dst, jnp.full((pad,), N, jnp.int32)]).reshape(NW, NCH, CH)
    ones_rows = jnp.ones((CH, HID), jnp.float32)

    h = _tc0_call(x, W1)
    degp = _deg_call(dstp, ones_rows)
    d0 = degp[0, :N, 0:1]
    d1 = degp[1, :N, 0:1]

    g1, dinv = _tc1_call(h, d0, d1)
    p = _agg_call(g1, srcp, dstp)
    g2 = _tc2_call(g1, p[0, :N], p[1, :N], dinv, b1.reshape(1, HID))
    q = _agg_call(g2, srcp, dstp)
    out = _tc3_call(g2, q[0, :N], q[1, :N], dinv, W2, b2.reshape(1, OUT_C))
    return out


# speedup vs baseline: 55.4996x; 1.5335x over previous
import jax
import jax.numpy as jnp
from jax import lax
from jax.experimental import pallas as pl
from jax.experimental.pallas import tpu as pltpu
from jax.experimental.pallas import tpu_sc as plsc

N = 10000
E = 320000
IN_C = 128
HID = 16
OUT_C = 128

NC = 2
NS = 16
NW = NC * NS
CH = 128
NCH = 78
TCH = 16
NBUF = 6
NPAD = 10240
RPT = NPAD // NS

_MESH = plsc.VectorSubcoreMesh(
    core_axis_name="c", subcore_axis_name="s", num_cores=NC, num_subcores=NS)

_SC_PARAMS = pltpu.CompilerParams(use_tc_tiling_on_sc=False)


def _zero_stage(stage_v, nrows):
    z = jnp.zeros((16,), jnp.float32)

    @pl.loop(0, nrows)
    def _(i):
        stage_v[i, :] = z


def _deg_body(dst_hbm, dstt_hbm, ones_hbm, out_hbm,
              dstv, dsttv, ones_v, stage_v, acc, sem):
    cid = lax.axis_index("c")
    sid = lax.axis_index("s")
    tid = cid * NS + sid

    pltpu.sync_copy(dst_hbm.at[tid], dstv)
    pltpu.sync_copy(dstt_hbm.at[tid], dsttv)
    pltpu.sync_copy(ones_hbm, ones_v)
    _zero_stage(stage_v, RPT)
    pltpu.sync_copy(stage_v, acc.at[pl.ds(sid * RPT, RPT)])
    plsc.subcore_barrier()

    @pl.loop(0, NCH // NBUF)
    def _(i):
        for b in range(NBUF):
            pltpu.async_copy(ones_v, acc.at[dstv.at[i * NBUF + b]], sem,
                             add=True)
        for b in range(NBUF):
            pltpu.make_async_copy(
                ones_v, acc.at[dstv.at[i * NBUF + b]], sem).wait()

    pltpu.sync_copy(ones_v.at[pl.ds(0, TCH)], acc.at[dsttv], add=True)

    plsc.subcore_barrier()
    pltpu.sync_copy(acc.at[pl.ds(sid * RPT, RPT)],
                    out_hbm.at[cid, pl.ds(sid * RPT, RPT)])


_deg_call = pl.kernel(
    _deg_body,
    out_type=jax.ShapeDtypeStruct((NC, NPAD, HID), jnp.float32),
    mesh=_MESH,
    scratch_types=[
        pltpu.VMEM((NCH, CH), jnp.int32),
        pltpu.VMEM((TCH,), jnp.int32),
        pltpu.VMEM((CH, HID), jnp.float32),
        pltpu.VMEM((RPT, HID), jnp.float32),
        pltpu.VMEM_SHARED((NPAD, HID), jnp.float32),
        pltpu.SemaphoreType.DMA,
    ],
    compiler_params=_SC_PARAMS,
)


def _agg_body(g_hbm, src_hbm, dst_hbm, srct_hbm, dstt_hbm, out_hbm,
              srcv, dstv, srctv, dsttv, rows, tail_v, stage_v, acc, *sems):
    gsems = sems[:NBUF]
    ssems = sems[NBUF:2 * NBUF]
    tsem = sems[2 * NBUF]
    cid = lax.axis_index("c")
    sid = lax.axis_index("s")
    tid = cid * NS + sid

    pltpu.sync_copy(src_hbm.at[tid], srcv)
    pltpu.sync_copy(dst_hbm.at[tid], dstv)
    pltpu.sync_copy(srct_hbm.at[tid], srctv)
    pltpu.sync_copy(dstt_hbm.at[tid], dsttv)
    _zero_stage(stage_v, RPT)
    pltpu.sync_copy(stage_v, acc.at[pl.ds(sid * RPT, RPT)])
    plsc.subcore_barrier()

    pltpu.async_copy(g_hbm.at[srctv], tail_v, tsem)
    for b in range(NBUF):
        pltpu.async_copy(g_hbm.at[srcv.at[b]], rows.at[b], gsems[b])
    pltpu.make_async_copy(g_hbm.at[srctv], tail_v, tsem).wait()
    pltpu.async_copy(tail_v, acc.at[dsttv], tsem, add=True)

    niter = NCH // NBUF

    @pl.loop(0, niter)
    def _(i):
        not_last = i + 1 < niter
        for b in range(NBUF):
            c = i * NBUF + b
            pltpu.make_async_copy(
                g_hbm.at[srcv.at[c]], rows.at[b], gsems[b]).wait()
            pltpu.async_copy(rows.at[b], acc.at[dstv.at[c]], ssems[b],
                             add=True)
        for b in range(NBUF):
            c = i * NBUF + b
            pltpu.make_async_copy(
                rows.at[b], acc.at[dstv.at[c]], ssems[b]).wait()

            @pl.when(not_last)
            def _():
                pltpu.async_copy(
                    g_hbm.at[srcv.at[c + NBUF]], rows.at[b], gsems[b])

    pltpu.make_async_copy(tail_v, acc.at[dsttv], tsem).wait()

    plsc.subcore_barrier()
    pltpu.sync_copy(acc.at[pl.ds(sid * RPT, RPT)],
                    out_hbm.at[cid, pl.ds(sid * RPT, RPT)])


_agg_call = pl.kernel(
    _agg_body,
    out_type=jax.ShapeDtypeStruct((NC, NPAD, HID), jnp.float32),
    mesh=_MESH,
    scratch_types=[
        pltpu.VMEM((NCH, CH), jnp.int32),
        pltpu.VMEM((NCH, CH), jnp.int32),
        pltpu.VMEM((TCH,), jnp.int32),
        pltpu.VMEM((TCH,), jnp.int32),
        pltpu.VMEM((NBUF, CH, HID), jnp.float32),
        pltpu.VMEM((TCH, HID), jnp.float32),
        pltpu.VMEM((RPT, HID), jnp.float32),
        pltpu.VMEM_SHARED((NPAD, HID), jnp.float32),
    ] + [pltpu.SemaphoreType.DMA] * (2 * NBUF + 1),
    compiler_params=_SC_PARAMS,
)


def _tc0_body(x_ref, w1_ref, h_ref):
    h_ref[...] = jnp.dot(x_ref[...], w1_ref[...],
                         preferred_element_type=jnp.float32)


def _tc1_body(h_ref, degp_ref, g1_ref, dinv_ref):
    deg = degp_ref[0, :N] + degp_ref[1, :N] + 1.0
    dinv = lax.rsqrt(deg)
    g1_ref[...] = h_ref[...] * dinv
    dinv_ref[...] = dinv


def _tc2_body(g1_ref, p_ref, dinv_ref, b1_ref, g2_ref):
    s = g1_ref[...] + p_ref[0, :N] + p_ref[1, :N]
    h1 = jnp.maximum(s * dinv_ref[...] + b1_ref[...], 0.0)
    g2_ref[...] = h1 * dinv_ref[...]


def _tc3_body(g2_ref, q_ref, dinv_ref, w2_ref, b2_ref, out_ref):
    s = (g2_ref[...] + q_ref[0, :N] + q_ref[1, :N]) * dinv_ref[...]
    out_ref[...] = (
        jnp.dot(s, w2_ref[...], preferred_element_type=jnp.float32)
        + b2_ref[...])


_tc0_call = pl.pallas_call(
    _tc0_body,
    out_shape=jax.ShapeDtypeStruct((N, HID), jnp.float32),
)

_tc1_call = pl.pallas_call(
    _tc1_body,
    out_shape=(jax.ShapeDtypeStruct((N, HID), jnp.float32),
               jax.ShapeDtypeStruct((N, HID), jnp.float32)),
)

_tc2_call = pl.pallas_call(
    _tc2_body,
    out_shape=jax.ShapeDtypeStruct((N, HID), jnp.float32),
)

_tc3_call = pl.pallas_call(
    _tc3_body,
    out_shape=jax.ShapeDtypeStruct((N, OUT_C), jnp.float32),
)


def kernel(x, A, W1, b1, W2, b2):
    main = NW * NCH * CH
    src = A[0].astype(jnp.int32)
    dst = A[1].astype(jnp.int32)
    srcm = src[:main].reshape(NW, NCH, CH)
    dstm = dst[:main].reshape(NW, NCH, CH)
    srct = src[main:].reshape(NW, TCH)
    dstt = dst[main:].reshape(NW, TCH)
    ones_rows = jnp.ones((CH, HID), jnp.float32)

    h = _tc0_call(x, W1)
    degp = _deg_call(dstm, dstt, ones_rows)

    g1, dinv = _tc1_call(h, degp)
    p = _agg_call(g1, srcm, dstm, srct, dstt)
    g2 = _tc2_call(g1, p, dinv, b1.reshape(1, HID))
    q = _agg_call(g2, srcm, dstm, srct, dstt)
    out = _tc3_call(g2, q, dinv, W2, b2.reshape(1, OUT_C))
    return out


# speedup vs baseline: 59.5309x; 1.0726x over previous
import jax
import jax.numpy as jnp
from jax import lax
from jax.experimental import pallas as pl
from jax.experimental.pallas import tpu as pltpu
from jax.experimental.pallas import tpu_sc as plsc

N = 10000
E = 320000
IN_C = 128
HID = 16
OUT_C = 128

NC = 2
NS = 16
NW = NC * NS
CH = 128
TOTCH = E // CH
NCH = 78
XTRA = TOTCH - NW * NCH
NBUF = 6
NPAD = 10240
RPT = NPAD // NS
PK = NPAD // 8
PKN = N // 8

_MESH = plsc.VectorSubcoreMesh(
    core_axis_name="c", subcore_axis_name="s", num_cores=NC, num_subcores=NS)

_SC_PARAMS = pltpu.CompilerParams(use_tc_tiling_on_sc=False)


def _zero_stage(stage_v, nrows):
    z = jnp.zeros((16,), jnp.float32)

    @pl.loop(0, nrows)
    def _(i):
        stage_v[i, :] = z


def _deg_body(dst_hbm, ones_hbm, out_hbm, dstv, ones_v, stage_v, acc, sem):
    cid = lax.axis_index("c")
    sid = lax.axis_index("s")
    tid = cid * NS + sid
    base = tid * NCH + jnp.minimum(tid, XTRA)

    @pl.when(tid < XTRA)
    def _():
        pltpu.sync_copy(dst_hbm.at[pl.ds(base, NCH + 1)], dstv)

    @pl.when(tid >= XTRA)
    def _():
        pltpu.sync_copy(dst_hbm.at[pl.ds(base, NCH)], dstv.at[pl.ds(0, NCH)])

    pltpu.sync_copy(ones_hbm, ones_v)
    _zero_stage(stage_v, RPT)
    pltpu.sync_copy(stage_v, acc.at[pl.ds(sid * RPT, RPT)])
    plsc.subcore_barrier()

    @pl.loop(0, NCH // NBUF)
    def _(i):
        for b in range(NBUF):
            pltpu.async_copy(ones_v, acc.at[dstv.at[i * NBUF + b]], sem,
                             add=True)
        for b in range(NBUF):
            pltpu.make_async_copy(
                ones_v, acc.at[dstv.at[i * NBUF + b]], sem).wait()

    @pl.when(tid < XTRA)
    def _():
        pltpu.sync_copy(ones_v, acc.at[dstv.at[NCH]], add=True)

    plsc.subcore_barrier()
    pltpu.sync_copy(acc.at[pl.ds(sid * RPT, RPT)],
                    out_hbm.at[cid, pl.ds(sid * RPT, RPT)])


_deg_call = pl.kernel(
    _deg_body,
    out_type=jax.ShapeDtypeStruct((NC, NPAD, HID), jnp.float32),
    mesh=_MESH,
    scratch_types=[
        pltpu.VMEM((NCH + 1, CH), jnp.int32),
        pltpu.VMEM((CH, HID), jnp.float32),
        pltpu.VMEM((RPT, HID), jnp.float32),
        pltpu.VMEM_SHARED((NPAD, HID), jnp.float32),
        pltpu.SemaphoreType.DMA,
    ],
    compiler_params=_SC_PARAMS,
)


def _agg_body(g_hbm, src_hbm, dst_hbm, out_hbm,
              srcv, dstv, rows, tail_v, stage_v, acc, *sems):
    gsems = sems[:NBUF]
    ssems = sems[NBUF:2 * NBUF]
    tsem = sems[2 * NBUF]
    cid = lax.axis_index("c")
    sid = lax.axis_index("s")
    tid = cid * NS + sid
    base = tid * NCH + jnp.minimum(tid, XTRA)
    has_xtra = tid < XTRA

    @pl.when(has_xtra)
    def _():
        pltpu.sync_copy(src_hbm.at[pl.ds(base, NCH + 1)], srcv)
        pltpu.sync_copy(dst_hbm.at[pl.ds(base, NCH + 1)], dstv)

    @pl.when(tid >= XTRA)
    def _():
        pltpu.sync_copy(src_hbm.at[pl.ds(base, NCH)], srcv.at[pl.ds(0, NCH)])
        pltpu.sync_copy(dst_hbm.at[pl.ds(base, NCH)], dstv.at[pl.ds(0, NCH)])

    _zero_stage(stage_v, RPT)
    pltpu.sync_copy(stage_v, acc.at[pl.ds(sid * RPT, RPT)])
    plsc.subcore_barrier()

    @pl.when(has_xtra)
    def _():
        pltpu.async_copy(g_hbm.at[srcv.at[NCH]], tail_v, tsem)

    for b in range(NBUF):
        pltpu.async_copy(g_hbm.at[srcv.at[b]], rows.at[b], gsems[b])

    @pl.when(has_xtra)
    def _():
        pltpu.make_async_copy(g_hbm.at[srcv.at[NCH]], tail_v, tsem).wait()
        pltpu.async_copy(tail_v, acc.at[dstv.at[NCH]], tsem, add=True)

    niter = NCH // NBUF

    @pl.loop(0, niter)
    def _(i):
        not_last = i + 1 < niter
        for b in range(NBUF):
            c = i * NBUF + b
            pltpu.make_async_copy(
                g_hbm.at[srcv.at[c]], rows.at[b], gsems[b]).wait()
            pltpu.async_copy(rows.at[b], acc.at[dstv.at[c]], ssems[b],
                             add=True)
        for b in range(NBUF):
            c = i * NBUF + b
            pltpu.make_async_copy(
                rows.at[b], acc.at[dstv.at[c]], ssems[b]).wait()

            @pl.when(not_last)
            def _():
                pltpu.async_copy(
                    g_hbm.at[srcv.at[c + NBUF]], rows.at[b], gsems[b])

    @pl.when(has_xtra)
    def _():
        pltpu.make_async_copy(tail_v, acc.at[dstv.at[NCH]], tsem).wait()

    plsc.subcore_barrier()
    pltpu.sync_copy(acc.at[pl.ds(sid * RPT, RPT)],
                    out_hbm.at[cid, pl.ds(sid * RPT, RPT)])


_agg_call = pl.kernel(
    _agg_body,
    out_type=jax.ShapeDtypeStruct((NC, NPAD, HID), jnp.float32),
    mesh=_MESH,
    scratch_types=[
        pltpu.VMEM((NCH + 1, CH), jnp.int32),
        pltpu.VMEM((NCH + 1, CH), jnp.int32),
        pltpu.VMEM((NBUF, CH, HID), jnp.float32),
        pltpu.VMEM((CH, HID), jnp.float32),
        pltpu.VMEM((RPT, HID), jnp.float32),
        pltpu.VMEM_SHARED((NPAD, HID), jnp.float32),
    ] + [pltpu.SemaphoreType.DMA] * (2 * NBUF + 1),
    compiler_params=_SC_PARAMS,
)


def _tc0_body(x_ref, w1_ref, h_ref):
    h_ref[...] = jnp.dot(x_ref[...], w1_ref[...],
                         preferred_element_type=jnp.float32)


def _tc1_body(h_ref, degp_ref, g1_ref, dinv_ref):
    deg = degp_ref[0, :N] + degp_ref[1, :N] + 1.0
    dinv = lax.rsqrt(deg)
    g1_ref[...] = h_ref[...] * dinv
    dinv_ref[...] = dinv


def _tc2_body(g1_ref, p_ref, dinv_ref, b1_ref, g2_ref):
    s = g1_ref[...] + p_ref[0, :N] + p_ref[1, :N]
    h1 = jnp.maximum(s * dinv_ref[...] + b1_ref[...], 0.0)
    g2_ref[...] = h1 * dinv_ref[...]


def _tc3_body(g2_ref, q_ref, dinv_ref, w2_ref, b2_ref, out_ref):
    s = (g2_ref[...] + q_ref[0, :N] + q_ref[1, :N]) * dinv_ref[...]
    out_ref[...] = (
        jnp.dot(s, w2_ref[...], preferred_element_type=jnp.float32)
        + b2_ref[...])


_tc0_call = pl.pallas_call(
    _tc0_body,
    out_shape=jax.ShapeDtypeStruct((N, HID), jnp.float32),
)

_tc1_call = pl.pallas_call(
    _tc1_body,
    out_shape=(jax.ShapeDtypeStruct((N, HID), jnp.float32),
               jax.ShapeDtypeStruct((N, HID), jnp.float32)),
)

_tc2_call = pl.pallas_call(
    _tc2_body,
    out_shape=jax.ShapeDtypeStruct((N, HID), jnp.float32),
)

_tc3_call = pl.pallas_call(
    _tc3_body,
    out_shape=jax.ShapeDtypeStruct((N, OUT_C), jnp.float32),
)


def kernel(x, A, W1, b1, W2, b2):
    src = A[0].astype(jnp.int32).reshape(TOTCH, CH)
    dst = A[1].astype(jnp.int32).reshape(TOTCH, CH)
    ones_rows = jnp.ones((CH, HID), jnp.float32)

    h = _tc0_call(x, W1)
    degp = _deg_call(dst, ones_rows)

    g1, dinv = _tc1_call(h, degp)
    p = _agg_call(g1, src, dst)
    g2 = _tc2_call(g1, p, dinv, b1.reshape(1, HID))
    q = _agg_call(g2, src, dst)
    out = _tc3_call(g2, q, dinv, W2, b2.reshape(1, OUT_C))
    return out


# speedup vs baseline: 84.1140x; 1.4129x over previous
import jax
import jax.numpy as jnp
from jax import lax
from jax.experimental import pallas as pl
from jax.experimental.pallas import tpu as pltpu
from jax.experimental.pallas import tpu_sc as plsc

N = 10000
E = 320000
IN_C = 128
HID = 16
OUT_C = 128

NC = 2
NS = 16
NW = NC * NS
CH = 128
TOTCH = E // CH
NCH = 78
XTRA = TOTCH - NW * NCH
NBUF = 6
NPAD = 10240
RPT = NPAD // NS
PK = NPAD // 8
PKN = N // 8

_MESH = plsc.VectorSubcoreMesh(
    core_axis_name="c", subcore_axis_name="s", num_cores=NC, num_subcores=NS)

_SC_PARAMS = pltpu.CompilerParams(use_tc_tiling_on_sc=False)


def _zero_stage(stage_v, nrows):
    z = jnp.zeros((16,), jnp.float32)

    @pl.loop(0, nrows)
    def _(i):
        stage_v[i, :] = z


def _deg_body(dst_hbm, ones_hbm, out_hbm, dstv, ones_v, stage_v, acc, sem):
    cid = lax.axis_index("c")
    sid = lax.axis_index("s")
    tid = cid * NS + sid
    base = tid * NCH + jnp.minimum(tid, XTRA)

    @pl.when(tid < XTRA)
    def _():
        pltpu.sync_copy(dst_hbm.at[pl.ds(base, NCH + 1)], dstv)

    @pl.when(tid >= XTRA)
    def _():
        pltpu.sync_copy(dst_hbm.at[pl.ds(base, NCH)], dstv.at[pl.ds(0, NCH)])

    pltpu.sync_copy(ones_hbm, ones_v)
    _zero_stage(stage_v, RPT)
    pltpu.sync_copy(stage_v, acc.at[pl.ds(sid * RPT, RPT)])
    plsc.subcore_barrier()

    @pl.loop(0, NCH // NBUF)
    def _(i):
        for b in range(NBUF):
            pltpu.async_copy(ones_v, acc.at[dstv.at[i * NBUF + b]], sem,
                             add=True)
        for b in range(NBUF):
            pltpu.make_async_copy(
                ones_v, acc.at[dstv.at[i * NBUF + b]], sem).wait()

    @pl.when(tid < XTRA)
    def _():
        pltpu.sync_copy(ones_v, acc.at[dstv.at[NCH]], add=True)

    plsc.subcore_barrier()
    pltpu.sync_copy(acc.at[pl.ds(sid * RPT, RPT)],
                    out_hbm.at[cid, pl.ds(sid * RPT, RPT)])


_deg_call = pl.kernel(
    _deg_body,
    out_type=jax.ShapeDtypeStruct((NC, NPAD, HID), jnp.float32),
    mesh=_MESH,
    scratch_types=[
        pltpu.VMEM((NCH + 1, CH), jnp.int32),
        pltpu.VMEM((CH, HID), jnp.float32),
        pltpu.VMEM((RPT, HID), jnp.float32),
        pltpu.VMEM_SHARED((NPAD, HID), jnp.float32),
        pltpu.SemaphoreType.DMA,
    ],
    compiler_params=_SC_PARAMS,
)


def _agg_body(g_hbm, src_hbm, dst_hbm, out_hbm,
              srcv, dstv, rows, tail_v, stage_v, acc, *sems):
    gsems = sems[:NBUF]
    ssems = sems[NBUF:2 * NBUF]
    tsem = sems[2 * NBUF]
    cid = lax.axis_index("c")
    sid = lax.axis_index("s")
    tid = cid * NS + sid
    base = tid * NCH + jnp.minimum(tid, XTRA)
    has_xtra = tid < XTRA

    @pl.when(has_xtra)
    def _():
        pltpu.sync_copy(src_hbm.at[pl.ds(base, NCH + 1)], srcv)
        pltpu.sync_copy(dst_hbm.at[pl.ds(base, NCH + 1)], dstv)

    @pl.when(tid >= XTRA)
    def _():
        pltpu.sync_copy(src_hbm.at[pl.ds(base, NCH)], srcv.at[pl.ds(0, NCH)])
        pltpu.sync_copy(dst_hbm.at[pl.ds(base, NCH)], dstv.at[pl.ds(0, NCH)])

    _zero_stage(stage_v, RPT)
    pltpu.sync_copy(stage_v, acc.at[pl.ds(sid * RPT, RPT)])
    plsc.subcore_barrier()

    @pl.when(has_xtra)
    def _():
        pltpu.async_copy(g_hbm.at[srcv.at[NCH]], tail_v, tsem)

    for b in range(NBUF):
        pltpu.async_copy(g_hbm.at[srcv.at[b]], rows.at[b], gsems[b])

    @pl.when(has_xtra)
    def _():
        pltpu.make_async_copy(g_hbm.at[srcv.at[NCH]], tail_v, tsem).wait()
        pltpu.async_copy(tail_v, acc.at[dstv.at[NCH]], tsem, add=True)

    niter = NCH // NBUF

    @pl.loop(0, niter)
    def _(i):
        not_last = i + 1 < niter
        for b in range(NBUF):
            c = i * NBUF + b
            pltpu.make_async_copy(
                g_hbm.at[srcv.at[c]], rows.at[b], gsems[b]).wait()
            pltpu.async_copy(rows.at[b], acc.at[dstv.at[c]], ssems[b],
                             add=True)
        for b in range(NBUF):
            c = i * NBUF + b
            pltpu.make_async_copy(
                rows.at[b], acc.at[dstv.at[c]], ssems[b]).wait()

            @pl.when(not_last)
            def _():
                pltpu.async_copy(
                    g_hbm.at[srcv.at[c + NBUF]], rows.at[b], gsems[b])

    @pl.when(has_xtra)
    def _():
        pltpu.make_async_copy(tail_v, acc.at[dstv.at[NCH]], tsem).wait()

    plsc.subcore_barrier()
    pltpu.sync_copy(acc.at[pl.ds(sid * RPT, RPT)],
                    out_hbm.at[cid, pl.ds(sid * RPT, RPT)])


_agg_call = pl.kernel(
    _agg_body,
    out_type=jax.ShapeDtypeStruct((NC, NPAD, HID), jnp.float32),
    mesh=_MESH,
    scratch_types=[
        pltpu.VMEM((NCH + 1, CH), jnp.int32),
        pltpu.VMEM((NCH + 1, CH), jnp.int32),
        pltpu.VMEM((NBUF, CH, HID), jnp.float32),
        pltpu.VMEM((CH, HID), jnp.float32),
        pltpu.VMEM((RPT, HID), jnp.float32),
        pltpu.VMEM_SHARED((NPAD, HID), jnp.float32),
    ] + [pltpu.SemaphoreType.DMA] * (2 * NBUF + 1),
    compiler_params=_SC_PARAMS,
)


def _tc0_body(xr_ref, w1e_ref, hp_ref):
    hp = jnp.dot(xr_ref[:, 0, :], w1e_ref[0],
                 preferred_element_type=jnp.float32)
    for a in range(1, 8):
        hp = hp + jnp.dot(xr_ref[:, a, :], w1e_ref[a],
                          preferred_element_type=jnp.float32)
    hp_ref[...] = jnp.concatenate(
        [hp, jnp.zeros((PK - PKN, 128), jnp.float32)], axis=0)


def _tc1_body(hp_ref, degp_ref, g1_ref, dinv_ref):
    deg = degp_ref[0] + degp_ref[1] + 1.0
    dinv = lax.rsqrt(deg)
    g1_ref[...] = hp_ref[...] * dinv
    dinv_ref[...] = dinv


def _tc2_body(g1_ref, p_ref, dinv_ref, b1_ref, g2_ref):
    s = g1_ref[...] + p_ref[0] + p_ref[1]
    h1 = jnp.maximum(s * dinv_ref[...] + b1_ref[...], 0.0)
    g2_ref[...] = h1 * dinv_ref[...]


def _tc3_body(g2_ref, q_ref, dinv_ref, w2e_ref, b2_ref, out_ref):
    s = (g2_ref[...] + q_ref[0] + q_ref[1]) * dinv_ref[...]
    sp = s[:PKN, :]
    for a in range(8):
        out_ref[:, a, :] = (
            jnp.dot(sp, w2e_ref[a], preferred_element_type=jnp.float32)
            + b2_ref[...])


_tc0_call = pl.pallas_call(
    _tc0_body,
    out_shape=jax.ShapeDtypeStruct((PK, 128), jnp.float32),
)

_tc1_call = pl.pallas_call(
    _tc1_body,
    out_shape=(jax.ShapeDtypeStruct((PK, 128), jnp.float32),
               jax.ShapeDtypeStruct((PK, 128), jnp.float32)),
)

_tc2_call = pl.pallas_call(
    _tc2_body,
    out_shape=jax.ShapeDtypeStruct((PK, 128), jnp.float32),
)

_tc3_call = pl.pallas_call(
    _tc3_body,
    out_shape=jax.ShapeDtypeStruct((PKN, 8, OUT_C), jnp.float32),
)


def kernel(x, A, W1, b1, W2, b2):
    src = A[0].astype(jnp.int32).reshape(TOTCH, CH)
    dst = A[1].astype(jnp.int32).reshape(TOTCH, CH)
    ones_rows = jnp.ones((CH, HID), jnp.float32)
    b1p = jnp.tile(b1, 8).reshape(1, 128)

    blk = jnp.arange(8 * HID, dtype=jnp.int32) // HID
    a_ids = jnp.arange(8, dtype=jnp.int32)
    onehot = (blk[None, :] == a_ids[:, None]).astype(jnp.float32)
    w1e = onehot[:, None, :] * jnp.tile(W1, (1, 8))[None]
    w2e = onehot[:, :, None] * jnp.tile(W2, (8, 1))[None]

    xr = x.reshape(PKN, 8, IN_C)
    hp = _tc0_call(xr, w1e)
    degp = _deg_call(dst, ones_rows)

    g1, dinv = _tc1_call(hp, degp.reshape(NC, PK, 128))
    p = _agg_call(g1.reshape(NPAD, HID), src, dst)
    g2 = _tc2_call(g1, p.reshape(NC, PK, 128), dinv, b1p)
    q = _agg_call(g2.reshape(NPAD, HID), src, dst)
    out = _tc3_call(g2, q.reshape(NC, PK, 128), dinv, w2e,
                    b2.reshape(1, OUT_C))
    return out.reshape(N, OUT_C)
